# Initial kernel scaffold; baseline (speedup 1.0000x reference)
import functools

import jax
import jax.numpy as jnp
from jax import lax
from jax.experimental import pallas as pl
from jax.experimental.pallas import tpu as pltpu
from jax.experimental.pallas import tpu_sc as plsc

N_ROWS = 10000
DIM = 256
N_EDGES = 160000
B_PAIR = 16384
ALPHA = 0.1

NC, NS = 2, 16
SROWS = 10240
STRIPE = SROWS // NS
E_PAD = 163840
NCHUNK = E_PAD // NS // 128
GCHUNK = B_PAIR // (NC * NS) // 128


def _ln(x, g, b, eps=1e-5):
    mu = jnp.mean(x, axis=-1, keepdims=True)
    var = jnp.mean((x - mu) ** 2, axis=-1, keepdims=True)
    return (x - mu) * lax.rsqrt(var + eps) * g + b


def _swish(x):
    return x * jax.nn.sigmoid(x)


def _dot(a, b):
    return jnp.dot(a, b, preferred_element_type=jnp.float32)


def _prep_body(drug_ref, cell_ref, Wd, bd, Wc, bc, We, be,
               ng0, nb0, W10, b10, W20, b20,
               ng1, nb1, W11, b11, W21, b21,
               cng, cnb, d0_ref, c0_ref, cx_ref):
    dr = drug_ref[...]
    ce = cell_ref[...]
    d0_ref[...] = _dot(dr, Wd[...]) + bd[...]
    c0_ref[...] = _dot(ce, Wc[...]) + bc[...]
    x = _dot(ce, We[...]) + be[...]
    for ng, nb, W1, b1, W2, b2 in ((ng0, nb0, W10, b10, W20, b20),
                                   (ng1, nb1, W11, b11, W21, b21)):
        h = _ln(x, ng[...], nb[...])
        h = _dot(h, W1[...]) + b1[...]
        h = h[:, :2 * DIM] * jax.nn.sigmoid(h[:, 2 * DIM:])
        x = x + _dot(h, W2[...]) + b2[...]
    cx_ref[...] = _ln(x, cng[...], cnb[...])


def _prep_call(drug, cell, p):
    R = 2000
    full = lambda s: pl.BlockSpec(s, lambda i: (0, 0))
    rows = pl.BlockSpec((R, DIM), lambda i: (i, 0))
    w = full((DIM, DIM))
    v = full((1, DIM))
    in_specs = [rows, rows, w, v, w, v, w, v]
    for _ in range(2):
        in_specs += [v, v, full((DIM, 4 * DIM)), full((1, 4 * DIM)),
                     full((2 * DIM, DIM)), v]
    in_specs += [v, v]
    out = jax.ShapeDtypeStruct((N_ROWS, DIM), jnp.float32)
    args = [drug, cell, p['Wd'], p['bd'][None], p['Wc'], p['bc'][None],
            p['We'], p['be'][None]]
    for i in range(2):
        args += [p[f'b{i}_ng'][None], p[f'b{i}_nb'][None], p[f'b{i}_W1'],
                 p[f'b{i}_b1'][None], p[f'b{i}_W2'], p[f'b{i}_b2'][None]]
    args += [p['cn_g'][None], p['cn_b'][None]]
    return pl.pallas_call(
        _prep_body,
        grid=(N_ROWS // R,),
        in_specs=in_specs,
        out_specs=[rows, rows, rows],
        out_shape=[out, out, out],
    )(*args)


def _fill(ref, nrows, ncols, value):
    v = jnp.full((16,), value, ref.dtype)

    def body(i, _):
        for c in range(ncols // 16):
            ref[i, pl.ds(c * 16, 16)] = v
        return 0

    lax.fori_loop(0, nrows, body, 0)


def _spmm_body(ni_hbm, ei_hbm, tab_hbm, S_out, deg_out,
               ni_v, ei_v, rows_v, b16_v, S_sh, deg_sh, sem):
    cid = lax.axis_index("c")
    sid = lax.axis_index("s")
    base = sid * STRIPE

    _fill(rows_v, 128, 128, 0.0)
    _fill(b16_v, 128, 16, 0.0)

    def zbody(k, _):
        pltpu.sync_copy(rows_v, S_sh.at[pl.ds(base + k * 128, 128)])
        pltpu.sync_copy(b16_v, deg_sh.at[pl.ds(base + k * 128, 128)])
        return 0

    lax.fori_loop(0, STRIPE // 128, zbody, 0)
    plsc.subcore_barrier()

    pltpu.sync_copy(ni_hbm.at[sid], ni_v)
    pltpu.sync_copy(ei_hbm.at[sid], ei_v)

    def xbody(i, _):
        for c in range(8):
            u = ni_v[i, pl.ds(c * 16, 16)]
            ni_v[i, pl.ds(c * 16, 16)] = u + u + cid
        return 0

    lax.fori_loop(0, NCHUNK, xbody, 0)
    _fill(b16_v, 128, 16, 1.0)

    def chunk(j, _):
        pltpu.async_copy(tab_hbm.at[ni_v.at[j]], rows_v, sem).wait()
        pltpu.sync_copy(rows_v, S_sh.at[ei_v.at[j]], add=True)

        @pl.when(cid == 0)
        def _():
            pltpu.sync_copy(b16_v, deg_sh.at[ei_v.at[j]], add=True)

        return 0

    lax.fori_loop(0, NCHUNK, chunk, 0)
    plsc.subcore_barrier()

    def obody(k, _):
        pltpu.sync_copy(S_sh.at[pl.ds(base + k * 128, 128)], rows_v)
        pltpu.sync_copy(rows_v, S_out.at[cid, pl.ds(base + k * 128, 128)])

        @pl.when(cid == 0)
        def _():
            pltpu.sync_copy(deg_sh.at[pl.ds(base + k * 128, 128)], b16_v)
            pltpu.sync_copy(b16_v, deg_out.at[pl.ds(base + k * 128, 128)])

        return 0

    lax.fori_loop(0, STRIPE // 128, obody, 0)


@functools.partial(
    pl.kernel,
    out_type=[jax.ShapeDtypeStruct((NC, SROWS, 128), jnp.float32),
              jax.ShapeDtypeStruct((SROWS, 16), jnp.float32)],
    mesh=plsc.VectorSubcoreMesh(core_axis_name="c", subcore_axis_name="s"),
    scratch_types=[
        pltpu.VMEM((NCHUNK, 128), jnp.int32),
        pltpu.VMEM((NCHUNK, 128), jnp.int32),
        pltpu.VMEM((128, 128), jnp.float32),
        pltpu.VMEM((128, 16), jnp.float32),
        pltpu.VMEM_SHARED((SROWS, 128), jnp.float32),
        pltpu.VMEM_SHARED((SROWS, 16), jnp.float32),
        pltpu.SemaphoreType.DMA,
    ],
)
def _spmm_call(ni_hbm, ei_hbm, tab_hbm, S_out, deg_out,
               ni_v, ei_v, rows_v, b16_v, S_sh, deg_sh, sem):
    _spmm_body(ni_hbm, ei_hbm, tab_hbm, S_out, deg_out,
               ni_v, ei_v, rows_v, b16_v, S_sh, deg_sh, sem)


def _head_body(Slp, Srp, Sln, Srn, degp, degn, d0_ref,
               W0, W1, bb0, bb1, g, b, ndp_ref, ndn_ref):
    d0 = d0_ref[...]

    def head(Sl, Sr, deg, out_ref):
        bd = jnp.where(deg > 0, lax.rsqrt(jnp.maximum(deg, 1e-9)), 0.0)
        sl = Sl[...] * bd
        sr = Sr[...] * bd
        nd = d0
        for W, bias in ((W0, bb0), (W1, bb1)):
            ed = _dot(sl, W[:128]) + _dot(sr, W[128:]) + bias[...]
            nd = _ln(_swish(ed + ALPHA * nd), g[...], b[...])
        out_ref[...] = nd

    head(Slp, Srp, degp[...], ndp_ref)
    head(Sln, Srn, degn[...], ndn_ref)


def _head_call(Sp, Sn, degp, degn, d0, p):
    R = 2000
    half = pl.BlockSpec((R, 128), lambda i: (i, 0))
    rows = pl.BlockSpec((R, DIM), lambda i: (i, 0))
    dcol = pl.BlockSpec((R, 1), lambda i: (i, 0))
    full = lambda s: pl.BlockSpec(s, lambda i: (0, 0))
    out = jax.ShapeDtypeStruct((N_ROWS, DIM), jnp.float32)
    return pl.pallas_call(
        _head_body,
        grid=(N_ROWS // R,),
        in_specs=[half, half, half, half, dcol, dcol, rows,
                  full((DIM, DIM)), full((DIM, DIM)),
                  full((1, DIM)), full((1, DIM)),
                  full((1, DIM)), full((1, DIM))],
        out_specs=[rows, rows],
        out_shape=[out, out],
    )(Sp[0, :N_ROWS], Sp[1, :N_ROWS], Sn[0, :N_ROWS], Sn[1, :N_ROWS],
      degp[:N_ROWS, :1], degn[:N_ROWS, :1], d0,
      p['hWE0'], p['hWE1'], p['hbE0'][None], p['hbE1'][None],
      p['ln_g'][None], p['ln_b'][None])


def _gather_body(ndp_hbm, ndn_hbm, cx_hbm, idxd_hbm, idxc_hbm,
                 gd_out, gc_out, idxd_v, idxc_v, rows_v, sem):
    cid = lax.axis_index("c")
    sid = lax.axis_index("s")
    wid = sid * NC + cid
    base = wid * (GCHUNK * 128)
    pltpu.sync_copy(idxd_hbm.at[wid], idxd_v)
    pltpu.sync_copy(idxc_hbm.at[wid], idxc_v)

    def body(k, _):
        off = base + k * 128
        pltpu.async_copy(ndp_hbm.at[idxd_v.at[k]], rows_v, sem).wait()
        pltpu.sync_copy(rows_v, gd_out.at[0, pl.ds(off, 128)])
        pltpu.async_copy(ndn_hbm.at[idxd_v.at[k]], rows_v, sem).wait()
        pltpu.sync_copy(rows_v, gd_out.at[1, pl.ds(off, 128)])
        pltpu.async_copy(cx_hbm.at[idxc_v.at[k]], rows_v, sem).wait()
        pltpu.sync_copy(rows_v, gc_out.at[pl.ds(off, 128)])
        return 0

    lax.fori_loop(0, GCHUNK, body, 0)


@functools.partial(
    pl.kernel,
    out_type=[jax.ShapeDtypeStruct((2, B_PAIR, DIM), jnp.float32),
              jax.ShapeDtypeStruct((B_PAIR, DIM), jnp.float32)],
    mesh=plsc.VectorSubcoreMesh(core_axis_name="c", subcore_axis_name="s"),
    scratch_types=[
        pltpu.VMEM((GCHUNK, 128), jnp.int32),
        pltpu.VMEM((GCHUNK, 128), jnp.int32),
        pltpu.VMEM((128, DIM), jnp.float32),
        pltpu.SemaphoreType.DMA,
    ],
)
def _gather_call(ndp_hbm, ndn_hbm, cx_hbm, idxd_hbm, idxc_hbm,
                 gd_out, gc_out, idxd_v, idxc_v, rows_v, sem):
    _gather_body(ndp_hbm, ndn_hbm, cx_hbm, idxd_hbm, idxc_hbm,
                 gd_out, gc_out, idxd_v, idxc_v, rows_v, sem)


def _proj_body(gd_ref, gc_ref, g0, b0, W0, bb0, g1, b1, W1, bb1, W2, bb2,
               out_ref):
    xd = gd_ref[0]
    xc = gc_ref[...]
    mu = (jnp.sum(xd, -1, keepdims=True) + jnp.sum(xc, -1, keepdims=True)) / 512.0
    var = (jnp.sum((xd - mu) ** 2, -1, keepdims=True)
           + jnp.sum((xc - mu) ** 2, -1, keepdims=True)) / 512.0
    inv = lax.rsqrt(var + 1e-5)
    hd = _swish((xd - mu) * inv * g0[:, :DIM] + b0[:, :DIM])
    hc = _swish((xc - mu) * inv * g0[:, DIM:] + b0[:, DIM:])
    t = _dot(hd, W0[:DIM]) + _dot(hc, W0[DIM:]) + bb0[...]
    t = _swish(_ln(t, g1[...], b1[...]))
    t = _dot(t, W1[...]) + bb1[...]
    out_ref[0] = _dot(t, W2[...]) + bb2[...]


def _proj_call(gd, gc, p):
    R = 2048
    full = lambda s: pl.BlockSpec(s, lambda g, i: (0, 0))
    return pl.pallas_call(
        _proj_body,
        grid=(2, B_PAIR // R),
        in_specs=[pl.BlockSpec((1, R, DIM), lambda g, i: (g, i, 0)),
                  pl.BlockSpec((R, DIM), lambda g, i: (i, 0)),
                  full((1, 512)), full((1, 512)), full((512, 512)),
                  full((1, 512)), full((1, 512)), full((1, 512)),
                  full((512, DIM)), full((1, DIM)),
                  full((DIM, 2)), full((1, 2))],
        out_specs=pl.BlockSpec((1, R, 2), lambda g, i: (g, i, 0)),
        out_shape=jax.ShapeDtypeStruct((2, B_PAIR, 2), jnp.float32),
    )(gd, gc, p['p_ln0_g'][None], p['p_ln0_b'][None], p['p_W0'],
      p['p_b0'][None], p['p_ln1_g'][None], p['p_ln1_b'][None],
      p['p_W1'], p['p_b1'][None], p['p_W2'], p['p_b2'][None])


def kernel(drug_feature, hyperedge_pos_index, hyperedge_neg_index,
           cell_feature, pair, params):
    p = params
    d0, c0, cellx = _prep_call(drug_feature, cell_feature, p)
    tab = c0.reshape(2 * N_ROWS, 128)

    def run_spmm(he):
        ni = he[0].astype(jnp.int32)
        ei = he[1].astype(jnp.int32)
        pad = E_PAD - N_EDGES
        nia = jnp.concatenate([ni, jnp.zeros((pad,), jnp.int32)])
        eia = jnp.concatenate([ei, jnp.full((pad,), N_ROWS, jnp.int32)])
        return _spmm_call(nia.reshape(NS, NCHUNK, 128),
                          eia.reshape(NS, NCHUNK, 128), tab)

    Sp, degp = run_spmm(hyperedge_pos_index)
    Sn, degn = run_spmm(hyperedge_neg_index)
    ndp, ndn = _head_call(Sp, Sn, degp, degn, d0, p)

    idxd = pair[:, 1].astype(jnp.int32).reshape(NC * NS, GCHUNK, 128)
    idxc = pair[:, 0].astype(jnp.int32).reshape(NC * NS, GCHUNK, 128)
    gd, gc = _gather_call(ndp, ndn, cellx, idxd, idxc)
    out = _proj_call(gd, gc, p)
    return out[0], out[1]


# speedup vs baseline: 6.8363x; 6.8363x over previous
import functools

import jax
import jax.numpy as jnp
from jax import lax
from jax.experimental import pallas as pl
from jax.experimental.pallas import tpu as pltpu
from jax.experimental.pallas import tpu_sc as plsc

N_ROWS = 10000
DIM = 256
N_EDGES = 160000
B_PAIR = 16384
ALPHA = 0.1

NC, NS = 2, 16
SROWS = 10240
STRIPE = SROWS // NS
E_PAD = 163840
NCHUNK = E_PAD // NS // 128
GCHUNK = B_PAIR // (NC * NS) // 128


def _ln(x, g, b, eps=1e-5):
    mu = jnp.mean(x, axis=-1, keepdims=True)
    var = jnp.mean((x - mu) ** 2, axis=-1, keepdims=True)
    return (x - mu) * lax.rsqrt(var + eps) * g + b


def _swish(x):
    return x * jax.nn.sigmoid(x)


def _dot(a, b):
    return jnp.dot(a, b, preferred_element_type=jnp.float32)


def _prep_body(drug_ref, cell_ref, Wd, bd, Wc, bc, We, be,
               ng0, nb0, W10, b10, W20, b20,
               ng1, nb1, W11, b11, W21, b21,
               cng, cnb, d0_ref, c0_ref, cx_ref):
    dr = drug_ref[...]
    ce = cell_ref[...]
    d0_ref[...] = _dot(dr, Wd[...]) + bd[...]
    c0_ref[...] = _dot(ce, Wc[...]) + bc[...]
    x = _dot(ce, We[...]) + be[...]
    for ng, nb, W1, b1, W2, b2 in ((ng0, nb0, W10, b10, W20, b20),
                                   (ng1, nb1, W11, b11, W21, b21)):
        h = _ln(x, ng[...], nb[...])
        h = _dot(h, W1[...]) + b1[...]
        h = h[:, :2 * DIM] * jax.nn.sigmoid(h[:, 2 * DIM:])
        x = x + _dot(h, W2[...]) + b2[...]
    cx_ref[...] = _ln(x, cng[...], cnb[...])


def _prep_call(drug, cell, p):
    R = 2000
    full = lambda s: pl.BlockSpec(s, lambda i: (0, 0))
    rows = pl.BlockSpec((R, DIM), lambda i: (i, 0))
    w = full((DIM, DIM))
    v = full((1, DIM))
    in_specs = [rows, rows, w, v, w, v, w, v]
    for _ in range(2):
        in_specs += [v, v, full((DIM, 4 * DIM)), full((1, 4 * DIM)),
                     full((2 * DIM, DIM)), v]
    in_specs += [v, v]
    out = jax.ShapeDtypeStruct((N_ROWS, DIM), jnp.float32)
    args = [drug, cell, p['Wd'], p['bd'][None], p['Wc'], p['bc'][None],
            p['We'], p['be'][None]]
    for i in range(2):
        args += [p[f'b{i}_ng'][None], p[f'b{i}_nb'][None], p[f'b{i}_W1'],
                 p[f'b{i}_b1'][None], p[f'b{i}_W2'], p[f'b{i}_b2'][None]]
    args += [p['cn_g'][None], p['cn_b'][None]]
    return pl.pallas_call(
        _prep_body,
        grid=(N_ROWS // R,),
        in_specs=in_specs,
        out_specs=[rows, rows, rows],
        out_shape=[out, out, out],
    )(*args)


def _fill(ref, nrows, ncols, value):
    v = jnp.full((16,), value, ref.dtype)

    def body(i, _):
        for c in range(ncols // 16):
            ref[i, pl.ds(c * 16, 16)] = v
        return 0

    lax.fori_loop(0, nrows, body, 0)


def _spmm_body(ni_hbm, ei_hbm, tab_hbm, S_out,
               ni_v, ei_v, rows_v, S_sh, sem):
    cid = lax.axis_index("c")
    sid = lax.axis_index("s")
    base = sid * STRIPE

    _fill(rows_v, 128, 128, 0.0)

    def zbody(k, _):
        pltpu.sync_copy(rows_v, S_sh.at[pl.ds(base + k * 128, 128)])
        return 0

    lax.fori_loop(0, STRIPE // 128, zbody, 0)
    plsc.subcore_barrier()

    pltpu.sync_copy(ni_hbm.at[sid], ni_v)
    pltpu.sync_copy(ei_hbm.at[sid], ei_v)

    def xbody(i, _):
        for c in range(8):
            u = ni_v[i, pl.ds(c * 16, 16)]
            ni_v[i, pl.ds(c * 16, 16)] = u + u + cid
        return 0

    lax.fori_loop(0, NCHUNK, xbody, 0)

    def chunk(j, _):
        pltpu.async_copy(tab_hbm.at[ni_v.at[j]], rows_v, sem).wait()
        pltpu.sync_copy(rows_v, S_sh.at[ei_v.at[j]], add=True)
        return 0

    lax.fori_loop(0, NCHUNK, chunk, 0)
    plsc.subcore_barrier()

    def obody(k, _):
        pltpu.sync_copy(S_sh.at[pl.ds(base + k * 128, 128)], rows_v)
        pltpu.sync_copy(rows_v, S_out.at[cid, pl.ds(base + k * 128, 128)])
        return 0

    lax.fori_loop(0, STRIPE // 128, obody, 0)


@functools.cache
def _spmm_kernel():
    return pl.kernel(
        _spmm_body,
        out_type=jax.ShapeDtypeStruct((NC, SROWS, 128), jnp.float32),
        mesh=plsc.VectorSubcoreMesh(core_axis_name="c", subcore_axis_name="s"),
        scratch_types=[
            pltpu.VMEM((NCHUNK, 128), jnp.int32),
            pltpu.VMEM((NCHUNK, 128), jnp.int32),
            pltpu.VMEM((128, 128), jnp.float32),
            pltpu.VMEM_SHARED((SROWS, 128), jnp.float32),
            pltpu.SemaphoreType.DMA,
        ],
    )


def _spmm_call(ni_hbm, ei_hbm, tab_hbm):
    return _spmm_kernel()(ni_hbm, ei_hbm, tab_hbm)


def _deg_body(e_hbm, deg_out, ei_v, ones_v, deg_sh):
    cid = lax.axis_index("c")
    sid = lax.axis_index("s")
    base = sid * STRIPE

    _fill(ones_v, 128, 128, 0.0)

    def zbody(k, _):
        pltpu.sync_copy(ones_v, deg_sh.at[pl.ds(base + k * 128, 128)])
        return 0

    lax.fori_loop(0, STRIPE // 128, zbody, 0)
    plsc.subcore_barrier()

    pltpu.sync_copy(e_hbm.at[cid, sid], ei_v)
    _fill(ones_v, 128, 128, 1.0)

    def chunk(j, _):
        pltpu.sync_copy(ones_v, deg_sh.at[ei_v.at[j]], add=True)
        return 0

    lax.fori_loop(0, NCHUNK, chunk, 0)
    plsc.subcore_barrier()

    def obody(k, _):
        pltpu.sync_copy(deg_sh.at[pl.ds(base + k * 128, 128)], ones_v)
        pltpu.sync_copy(ones_v, deg_out.at[cid, pl.ds(base + k * 128, 128)])
        return 0

    lax.fori_loop(0, STRIPE // 128, obody, 0)


@functools.cache
def _deg_kernel():
    return pl.kernel(
        _deg_body,
        out_type=jax.ShapeDtypeStruct((NC, SROWS, 128), jnp.float32),
        mesh=plsc.VectorSubcoreMesh(core_axis_name="c", subcore_axis_name="s"),
        scratch_types=[
            pltpu.VMEM((NCHUNK, 128), jnp.int32),
            pltpu.VMEM((128, 128), jnp.float32),
            pltpu.VMEM_SHARED((SROWS, 128), jnp.float32),
        ],
    )


def _deg_call(eip_hbm, ein_hbm):
    deg = _deg_kernel()(jnp.stack([eip_hbm, ein_hbm]))
    return deg[0], deg[1]


def _head_body(Slp, Srp, Sln, Srn, degp, degn, d0_ref,
               W0, W1, bb0, bb1, g, b, ndp_ref, ndn_ref):
    d0 = d0_ref[...]

    def head(Sl, Sr, deg, out_ref):
        bd = jnp.where(deg > 0, lax.rsqrt(jnp.maximum(deg, 1e-9)), 0.0)
        sl = Sl[...] * bd
        sr = Sr[...] * bd
        nd = d0
        for W, bias in ((W0, bb0), (W1, bb1)):
            ed = _dot(sl, W[:128]) + _dot(sr, W[128:]) + bias[...]
            nd = _ln(_swish(ed + ALPHA * nd), g[...], b[...])
        out_ref[...] = nd

    head(Slp, Srp, degp[...], ndp_ref)
    head(Sln, Srn, degn[...], ndn_ref)


def _head_call(Sp, Sn, degp, degn, d0, p):
    R = 2000
    half = pl.BlockSpec((R, 128), lambda i: (i, 0))
    rows = pl.BlockSpec((R, DIM), lambda i: (i, 0))
    dcol = pl.BlockSpec((R, 1), lambda i: (i, 0))
    full = lambda s: pl.BlockSpec(s, lambda i: (0, 0))
    out = jax.ShapeDtypeStruct((N_ROWS, DIM), jnp.float32)
    return pl.pallas_call(
        _head_body,
        grid=(N_ROWS // R,),
        in_specs=[half, half, half, half, dcol, dcol, rows,
                  full((DIM, DIM)), full((DIM, DIM)),
                  full((1, DIM)), full((1, DIM)),
                  full((1, DIM)), full((1, DIM))],
        out_specs=[rows, rows],
        out_shape=[out, out],
    )(Sp[0, :N_ROWS], Sp[1, :N_ROWS], Sn[0, :N_ROWS], Sn[1, :N_ROWS],
      degp[:N_ROWS, :1], degn[:N_ROWS, :1], d0,
      p['hWE0'], p['hWE1'], p['hbE0'][None], p['hbE1'][None],
      p['ln_g'][None], p['ln_b'][None])


def _gather_body(ndp_hbm, ndn_hbm, cx_hbm, idxd_hbm, idxc_hbm,
                 gd_out, gc_out, idxd_v, idxc_v, rows_v, sem):
    cid = lax.axis_index("c")
    sid = lax.axis_index("s")
    wid = sid * NC + cid
    base = wid * (GCHUNK * 128)
    pltpu.sync_copy(idxd_hbm.at[wid], idxd_v)
    pltpu.sync_copy(idxc_hbm.at[wid], idxc_v)

    def body(k, _):
        off = base + k * 128
        pltpu.async_copy(ndp_hbm.at[idxd_v.at[k]], rows_v, sem).wait()
        pltpu.sync_copy(rows_v, gd_out.at[0, pl.ds(off, 128)])
        pltpu.async_copy(ndn_hbm.at[idxd_v.at[k]], rows_v, sem).wait()
        pltpu.sync_copy(rows_v, gd_out.at[1, pl.ds(off, 128)])
        pltpu.async_copy(cx_hbm.at[idxc_v.at[k]], rows_v, sem).wait()
        pltpu.sync_copy(rows_v, gc_out.at[pl.ds(off, 128)])
        return 0

    lax.fori_loop(0, GCHUNK, body, 0)


@functools.cache
def _gather_kernel():
    return pl.kernel(
        _gather_body,
        out_type=[jax.ShapeDtypeStruct((2, B_PAIR, DIM), jnp.float32),
                  jax.ShapeDtypeStruct((B_PAIR, DIM), jnp.float32)],
        mesh=plsc.VectorSubcoreMesh(core_axis_name="c", subcore_axis_name="s"),
        scratch_types=[
            pltpu.VMEM((GCHUNK, 128), jnp.int32),
            pltpu.VMEM((GCHUNK, 128), jnp.int32),
            pltpu.VMEM((128, DIM), jnp.float32),
            pltpu.SemaphoreType.DMA,
        ],
    )


def _gather_call(ndp_hbm, ndn_hbm, cx_hbm, idxd_hbm, idxc_hbm):
    return _gather_kernel()(ndp_hbm, ndn_hbm, cx_hbm, idxd_hbm, idxc_hbm)


def _proj_body(gd_ref, gc_ref, g0, b0, W0, bb0, g1, b1, W1, bb1, W2, bb2,
               out_ref):
    xd = gd_ref[0]
    xc = gc_ref[...]
    mu = (jnp.sum(xd, -1, keepdims=True) + jnp.sum(xc, -1, keepdims=True)) / 512.0
    var = (jnp.sum((xd - mu) ** 2, -1, keepdims=True)
           + jnp.sum((xc - mu) ** 2, -1, keepdims=True)) / 512.0
    inv = lax.rsqrt(var + 1e-5)
    hd = _swish((xd - mu) * inv * g0[:, :DIM] + b0[:, :DIM])
    hc = _swish((xc - mu) * inv * g0[:, DIM:] + b0[:, DIM:])
    t = _dot(hd, W0[:DIM]) + _dot(hc, W0[DIM:]) + bb0[...]
    t = _swish(_ln(t, g1[...], b1[...]))
    t = _dot(t, W1[...]) + bb1[...]
    out_ref[0] = _dot(t, W2[...]) + bb2[...]


def _proj_call(gd, gc, p):
    R = 2048
    full = lambda s: pl.BlockSpec(s, lambda g, i: (0, 0))
    return pl.pallas_call(
        _proj_body,
        grid=(2, B_PAIR // R),
        in_specs=[pl.BlockSpec((1, R, DIM), lambda g, i: (g, i, 0)),
                  pl.BlockSpec((R, DIM), lambda g, i: (i, 0)),
                  full((1, 512)), full((1, 512)), full((512, 512)),
                  full((1, 512)), full((1, 512)), full((1, 512)),
                  full((512, DIM)), full((1, DIM)),
                  full((DIM, 2)), full((1, 2))],
        out_specs=pl.BlockSpec((1, R, 2), lambda g, i: (g, i, 0)),
        out_shape=jax.ShapeDtypeStruct((2, B_PAIR, 2), jnp.float32),
    )(gd, gc, p['p_ln0_g'][None], p['p_ln0_b'][None], p['p_W0'],
      p['p_b0'][None], p['p_ln1_g'][None], p['p_ln1_b'][None],
      p['p_W1'], p['p_b1'][None], p['p_W2'], p['p_b2'][None])


def kernel(drug_feature, hyperedge_pos_index, hyperedge_neg_index,
           cell_feature, pair, params):
    p = params
    d0, c0, cellx = _prep_call(drug_feature, cell_feature, p)
    tab = c0.reshape(2 * N_ROWS, 128)

    def pad_idx(v, fill):
        pad = E_PAD - N_EDGES
        a = jnp.concatenate([v.astype(jnp.int32),
                             jnp.full((pad,), fill, jnp.int32)])
        return a.reshape(NS, NCHUNK, 128)

    nip = pad_idx(hyperedge_pos_index[0], 0)
    eip = pad_idx(hyperedge_pos_index[1], N_ROWS)
    nin = pad_idx(hyperedge_neg_index[0], 0)
    ein = pad_idx(hyperedge_neg_index[1], N_ROWS)

    Sp = _spmm_call(nip, eip, tab)
    Sn = _spmm_call(nin, ein, tab)
    degp, degn = _deg_call(eip, ein)
    ndp, ndn = _head_call(Sp, Sn, degp, degn, d0, p)

    idxd = pair[:, 1].astype(jnp.int32).reshape(NC * NS, GCHUNK, 128)
    idxc = pair[:, 0].astype(jnp.int32).reshape(NC * NS, GCHUNK, 128)
    gd, gc = _gather_call(ndp, ndn, cellx, idxd, idxc)
    out = _proj_call(gd, gc, p)
    return out[0], out[1]


# speedup vs baseline: 7.7864x; 1.1390x over previous
import functools

import jax
import jax.numpy as jnp
from jax import lax
from jax.experimental import pallas as pl
from jax.experimental.pallas import tpu as pltpu
from jax.experimental.pallas import tpu_sc as plsc

N_ROWS = 10000
DIM = 256
N_EDGES = 160000
B_PAIR = 16384
ALPHA = 0.1

NC, NS = 2, 16
SROWS = 10240
STRIPE = SROWS // NS
E_PAD = 163840
NCHUNK = E_PAD // NS // 128
GCHUNK = B_PAIR // (NC * NS) // 128


def _ln(x, g, b, eps=1e-5):
    mu = jnp.mean(x, axis=-1, keepdims=True)
    var = jnp.mean((x - mu) ** 2, axis=-1, keepdims=True)
    return (x - mu) * lax.rsqrt(var + eps) * g + b


def _swish(x):
    return x * jax.nn.sigmoid(x)


def _dot(a, b):
    return jnp.dot(a, b, preferred_element_type=jnp.float32)


def _prep_body(drug_ref, cell_ref, Wd, bd, Wc, bc, We, be,
               ng0, nb0, W10, b10, W20, b20,
               ng1, nb1, W11, b11, W21, b21,
               cng, cnb, d0_ref, c0_ref, cx_ref):
    dr = drug_ref[...]
    ce = cell_ref[...]
    d0_ref[...] = _dot(dr, Wd[...]) + bd[...]
    c0_ref[...] = _dot(ce, Wc[...]) + bc[...]
    x = _dot(ce, We[...]) + be[...]
    for ng, nb, W1, b1, W2, b2 in ((ng0, nb0, W10, b10, W20, b20),
                                   (ng1, nb1, W11, b11, W21, b21)):
        h = _ln(x, ng[...], nb[...])
        h = _dot(h, W1[...]) + b1[...]
        h = h[:, :2 * DIM] * jax.nn.sigmoid(h[:, 2 * DIM:])
        x = x + _dot(h, W2[...]) + b2[...]
    cx_ref[...] = _ln(x, cng[...], cnb[...])


def _prep_call(drug, cell, p):
    R = 2000
    full = lambda s: pl.BlockSpec(s, lambda i: (0, 0))
    rows = pl.BlockSpec((R, DIM), lambda i: (i, 0))
    w = full((DIM, DIM))
    v = full((1, DIM))
    in_specs = [rows, rows, w, v, w, v, w, v]
    for _ in range(2):
        in_specs += [v, v, full((DIM, 4 * DIM)), full((1, 4 * DIM)),
                     full((2 * DIM, DIM)), v]
    in_specs += [v, v]
    out = jax.ShapeDtypeStruct((N_ROWS, DIM), jnp.float32)
    args = [drug, cell, p['Wd'], p['bd'][None], p['Wc'], p['bc'][None],
            p['We'], p['be'][None]]
    for i in range(2):
        args += [p[f'b{i}_ng'][None], p[f'b{i}_nb'][None], p[f'b{i}_W1'],
                 p[f'b{i}_b1'][None], p[f'b{i}_W2'], p[f'b{i}_b2'][None]]
    args += [p['cn_g'][None], p['cn_b'][None]]
    return pl.pallas_call(
        _prep_body,
        grid=(N_ROWS // R,),
        in_specs=in_specs,
        out_specs=[rows, rows, rows],
        out_shape=[out, out, out],
    )(*args)


def _fill(ref, nrows, ncols, value):
    v = jnp.full((16,), value, ref.dtype)

    def body(i, _):
        for c in range(ncols // 16):
            ref[i, pl.ds(c * 16, 16)] = v
        return 0

    lax.fori_loop(0, nrows, body, 0)


def _spmm_body(ni_hbm, ei_hbm, tab_hbm, S_out,
               ni_v, ei_v, rows_v, S_sh, sem):
    cid = lax.axis_index("c")
    sid = lax.axis_index("s")
    base = sid * STRIPE

    _fill(rows_v.at[0], 128, 128, 0.0)

    def zbody(k, _):
        pltpu.sync_copy(rows_v.at[0], S_sh.at[pl.ds(base + k * 128, 128)])
        return 0

    lax.fori_loop(0, STRIPE // 128, zbody, 0)
    plsc.subcore_barrier()

    NH = NCHUNK // 2
    for h in range(2):
        pltpu.sync_copy(ni_hbm.at[sid, pl.ds(h * NH, NH)], ni_v)
        pltpu.sync_copy(ei_hbm.at[sid, pl.ds(h * NH, NH)], ei_v)

        def xbody(i, _):
            for c in range(8):
                u = ni_v[i, pl.ds(c * 16, 16)]
                ni_v[i, pl.ds(c * 16, 16)] = u + u + cid
            return 0

        lax.fori_loop(0, NH, xbody, 0)

        def gat(j, b):
            return pltpu.async_copy(tab_hbm.at[ni_v.at[j]], rows_v.at[b], sem)

        def wait_one():
            pltpu.make_async_copy(tab_hbm.at[ni_v.at[0]], rows_v.at[0], sem).wait()

        gat(0, 0)

        def chunk2(jj, _):
            j0 = 2 * jj
            gat(j0 + 1, 1)
            wait_one()
            pltpu.sync_copy(rows_v.at[0], S_sh.at[ei_v.at[j0]], add=True)

            @pl.when(jj < NH // 2 - 1)
            def _():
                gat(j0 + 2, 0)

            wait_one()
            pltpu.sync_copy(rows_v.at[1], S_sh.at[ei_v.at[j0 + 1]], add=True)
            return 0

        lax.fori_loop(0, NH // 2, chunk2, 0)

    plsc.subcore_barrier()

    def obody(k, _):
        pltpu.sync_copy(S_sh.at[pl.ds(base + k * 128, 128)], rows_v.at[0])
        pltpu.sync_copy(rows_v.at[0], S_out.at[cid, pl.ds(base + k * 128, 128)])
        return 0

    lax.fori_loop(0, STRIPE // 128, obody, 0)


@functools.cache
def _spmm_kernel():
    return pl.kernel(
        _spmm_body,
        out_type=jax.ShapeDtypeStruct((NC, SROWS, 128), jnp.float32),
        mesh=plsc.VectorSubcoreMesh(core_axis_name="c", subcore_axis_name="s"),
        scratch_types=[
            pltpu.VMEM((NCHUNK // 2, 128), jnp.int32),
            pltpu.VMEM((NCHUNK // 2, 128), jnp.int32),
            pltpu.VMEM((2, 128, 128), jnp.float32),
            pltpu.VMEM_SHARED((SROWS, 128), jnp.float32),
            pltpu.SemaphoreType.DMA,
        ],
    )


def _spmm_call(ni_hbm, ei_hbm, tab_hbm):
    return _spmm_kernel()(ni_hbm, ei_hbm, tab_hbm)


def _deg_body(e_hbm, deg_out, ei_v, ones_v, deg_sh):
    cid = lax.axis_index("c")
    sid = lax.axis_index("s")
    base = sid * STRIPE

    _fill(ones_v, 128, 128, 0.0)

    def zbody(k, _):
        pltpu.sync_copy(ones_v, deg_sh.at[pl.ds(base + k * 128, 128)])
        return 0

    lax.fori_loop(0, STRIPE // 128, zbody, 0)
    plsc.subcore_barrier()

    pltpu.sync_copy(e_hbm.at[cid, sid], ei_v)
    _fill(ones_v, 128, 128, 1.0)

    def chunk(j, _):
        pltpu.sync_copy(ones_v, deg_sh.at[ei_v.at[j]], add=True)
        return 0

    lax.fori_loop(0, NCHUNK, chunk, 0)
    plsc.subcore_barrier()

    def obody(k, _):
        pltpu.sync_copy(deg_sh.at[pl.ds(base + k * 128, 128)], ones_v)
        pltpu.sync_copy(ones_v, deg_out.at[cid, pl.ds(base + k * 128, 128)])
        return 0

    lax.fori_loop(0, STRIPE // 128, obody, 0)


@functools.cache
def _deg_kernel():
    return pl.kernel(
        _deg_body,
        out_type=jax.ShapeDtypeStruct((NC, SROWS, 128), jnp.float32),
        mesh=plsc.VectorSubcoreMesh(core_axis_name="c", subcore_axis_name="s"),
        scratch_types=[
            pltpu.VMEM((NCHUNK, 128), jnp.int32),
            pltpu.VMEM((128, 128), jnp.float32),
            pltpu.VMEM_SHARED((SROWS, 128), jnp.float32),
        ],
    )


def _deg_call(eip_hbm, ein_hbm):
    deg = _deg_kernel()(jnp.stack([eip_hbm, ein_hbm]))
    return deg[0], deg[1]


def _head_body(Slp, Srp, Sln, Srn, degp, degn, d0_ref,
               W0, W1, bb0, bb1, g, b, ndp_ref, ndn_ref):
    d0 = d0_ref[...]

    def head(Sl, Sr, deg, out_ref):
        bd = jnp.where(deg > 0, lax.rsqrt(jnp.maximum(deg, 1e-9)), 0.0)
        sl = Sl[...] * bd
        sr = Sr[...] * bd
        nd = d0
        for W, bias in ((W0, bb0), (W1, bb1)):
            ed = _dot(sl, W[:128]) + _dot(sr, W[128:]) + bias[...]
            nd = _ln(_swish(ed + ALPHA * nd), g[...], b[...])
        out_ref[...] = nd

    head(Slp, Srp, degp[...], ndp_ref)
    head(Sln, Srn, degn[...], ndn_ref)


def _head_call(Sp, Sn, degp, degn, d0, p):
    R = 2000
    half = pl.BlockSpec((R, 128), lambda i: (i, 0))
    rows = pl.BlockSpec((R, DIM), lambda i: (i, 0))
    dcol = pl.BlockSpec((R, 1), lambda i: (i, 0))
    full = lambda s: pl.BlockSpec(s, lambda i: (0, 0))
    out = jax.ShapeDtypeStruct((N_ROWS, DIM), jnp.float32)
    return pl.pallas_call(
        _head_body,
        grid=(N_ROWS // R,),
        in_specs=[half, half, half, half, dcol, dcol, rows,
                  full((DIM, DIM)), full((DIM, DIM)),
                  full((1, DIM)), full((1, DIM)),
                  full((1, DIM)), full((1, DIM))],
        out_specs=[rows, rows],
        out_shape=[out, out],
    )(Sp[0, :N_ROWS], Sp[1, :N_ROWS], Sn[0, :N_ROWS], Sn[1, :N_ROWS],
      degp[:N_ROWS, :1], degn[:N_ROWS, :1], d0,
      p['hWE0'], p['hWE1'], p['hbE0'][None], p['hbE1'][None],
      p['ln_g'][None], p['ln_b'][None])


def _gather_body(ndp_hbm, ndn_hbm, cx_hbm, idxd_hbm, idxc_hbm,
                 gd_out, gc_out, idxd_v, idxc_v, rows_v, sem):
    cid = lax.axis_index("c")
    sid = lax.axis_index("s")
    wid = sid * NC + cid
    base = wid * (GCHUNK * 128)
    pltpu.sync_copy(idxd_hbm.at[wid], idxd_v)
    pltpu.sync_copy(idxc_hbm.at[wid], idxc_v)

    def body(k, _):
        off = base + k * 128
        pltpu.async_copy(ndp_hbm.at[idxd_v.at[k]], rows_v, sem).wait()
        pltpu.sync_copy(rows_v, gd_out.at[0, pl.ds(off, 128)])
        pltpu.async_copy(ndn_hbm.at[idxd_v.at[k]], rows_v, sem).wait()
        pltpu.sync_copy(rows_v, gd_out.at[1, pl.ds(off, 128)])
        pltpu.async_copy(cx_hbm.at[idxc_v.at[k]], rows_v, sem).wait()
        pltpu.sync_copy(rows_v, gc_out.at[pl.ds(off, 128)])
        return 0

    lax.fori_loop(0, GCHUNK, body, 0)


@functools.cache
def _gather_kernel():
    return pl.kernel(
        _gather_body,
        out_type=[jax.ShapeDtypeStruct((2, B_PAIR, DIM), jnp.float32),
                  jax.ShapeDtypeStruct((B_PAIR, DIM), jnp.float32)],
        mesh=plsc.VectorSubcoreMesh(core_axis_name="c", subcore_axis_name="s"),
        scratch_types=[
            pltpu.VMEM((GCHUNK, 128), jnp.int32),
            pltpu.VMEM((GCHUNK, 128), jnp.int32),
            pltpu.VMEM((128, DIM), jnp.float32),
            pltpu.SemaphoreType.DMA,
        ],
    )


def _gather_call(ndp_hbm, ndn_hbm, cx_hbm, idxd_hbm, idxc_hbm):
    return _gather_kernel()(ndp_hbm, ndn_hbm, cx_hbm, idxd_hbm, idxc_hbm)


def _proj_body(gd_ref, gc_ref, g0, b0, W0, bb0, g1, b1, W1, bb1, W2, bb2,
               out_ref):
    xd = gd_ref[0]
    xc = gc_ref[...]
    mu = (jnp.sum(xd, -1, keepdims=True) + jnp.sum(xc, -1, keepdims=True)) / 512.0
    var = (jnp.sum((xd - mu) ** 2, -1, keepdims=True)
           + jnp.sum((xc - mu) ** 2, -1, keepdims=True)) / 512.0
    inv = lax.rsqrt(var + 1e-5)
    hd = _swish((xd - mu) * inv * g0[:, :DIM] + b0[:, :DIM])
    hc = _swish((xc - mu) * inv * g0[:, DIM:] + b0[:, DIM:])
    t = _dot(hd, W0[:DIM]) + _dot(hc, W0[DIM:]) + bb0[...]
    t = _swish(_ln(t, g1[...], b1[...]))
    t = _dot(t, W1[...]) + bb1[...]
    out_ref[0] = _dot(t, W2[...]) + bb2[...]


def _proj_call(gd, gc, p):
    R = 2048
    full = lambda s: pl.BlockSpec(s, lambda g, i: (0, 0))
    return pl.pallas_call(
        _proj_body,
        grid=(2, B_PAIR // R),
        in_specs=[pl.BlockSpec((1, R, DIM), lambda g, i: (g, i, 0)),
                  pl.BlockSpec((R, DIM), lambda g, i: (i, 0)),
                  full((1, 512)), full((1, 512)), full((512, 512)),
                  full((1, 512)), full((1, 512)), full((1, 512)),
                  full((512, DIM)), full((1, DIM)),
                  full((DIM, 2)), full((1, 2))],
        out_specs=pl.BlockSpec((1, R, 2), lambda g, i: (g, i, 0)),
        out_shape=jax.ShapeDtypeStruct((2, B_PAIR, 2), jnp.float32),
    )(gd, gc, p['p_ln0_g'][None], p['p_ln0_b'][None], p['p_W0'],
      p['p_b0'][None], p['p_ln1_g'][None], p['p_ln1_b'][None],
      p['p_W1'], p['p_b1'][None], p['p_W2'], p['p_b2'][None])


def kernel(drug_feature, hyperedge_pos_index, hyperedge_neg_index,
           cell_feature, pair, params):
    p = params
    d0, c0, cellx = _prep_call(drug_feature, cell_feature, p)
    tab = c0.reshape(2 * N_ROWS, 128)

    def pad_idx(v, fill):
        pad = E_PAD - N_EDGES
        a = jnp.concatenate([v.astype(jnp.int32),
                             jnp.full((pad,), fill, jnp.int32)])
        return a.reshape(NS, NCHUNK, 128)

    nip = pad_idx(hyperedge_pos_index[0], 0)
    eip = pad_idx(hyperedge_pos_index[1], N_ROWS)
    nin = pad_idx(hyperedge_neg_index[0], 0)
    ein = pad_idx(hyperedge_neg_index[1], N_ROWS)

    Sp = _spmm_call(nip, eip, tab)
    Sn = _spmm_call(nin, ein, tab)
    degp, degn = _deg_call(eip, ein)
    ndp, ndn = _head_call(Sp, Sn, degp, degn, d0, p)

    idxd = pair[:, 1].astype(jnp.int32).reshape(NC * NS, GCHUNK, 128)
    idxc = pair[:, 0].astype(jnp.int32).reshape(NC * NS, GCHUNK, 128)
    gd, gc = _gather_call(ndp, ndn, cellx, idxd, idxc)
    out = _proj_call(gd, gc, p)
    return out[0], out[1]
